# baseline (speedup 1.0000x reference)
import functools

import jax
import jax.numpy as jnp
from jax import lax
from jax.experimental import pallas as pl
from jax.experimental.pallas import tpu as pltpu
from jax.experimental.pallas import tpu_sc as plsc

N = 10000
E = 320000
D = 512
NK = 2000
NEG = 0.2
CP = 2048
NPADN = 10240
EPAD = 327680
EPT = EPAD // 16
NPT = NPADN // 16
HASH = 1 << 18
SLOWPT = 128
K = 16 * SLOWPT
MSIZE = CP * CP + 16384
BISECT_ITERS = 24


def _p1_body(info_col, info_row, x_ref, att_ref, nm_ref, p_ref, q_ref):
    pid = pl.program_id(0)
    ii = info_col[...]
    iidx = pid * 1000 + lax.broadcasted_iota(jnp.int32, (1000, 1), 0)
    rank = jnp.zeros((1000, 1), jnp.float32)
    for c in range(10):
        jj = info_row[:, c * 1024:(c + 1) * 1024]
        jidx = c * 1024 + lax.broadcasted_iota(jnp.int32, (1, 1024), 1)
        gt = (jj > ii).astype(jnp.float32)
        tie = ((jj == ii) & (jidx < iidx)).astype(jnp.float32)
        rank = rank + jnp.sum(gt + tie, axis=1, keepdims=True)
    r32 = rank.astype(jnp.int32)
    nm_ref[...] = jnp.where(r32 < NK, r32, -1)
    pq = lax.dot_general(x_ref[...], att_ref[...], (((1,), (0,)), ((), ())),
                         preferred_element_type=jnp.float32)
    p_ref[...] = pq[:, 0:1]
    q_ref[...] = pq[:, 1:2]


def _p1(info_pad, x, att_w):
    info_col = info_pad.reshape(NPADN, 1)
    info_row = info_pad.reshape(1, NPADN)
    return pl.pallas_call(
        _p1_body,
        grid=(10,),
        in_specs=[
            pl.BlockSpec((1000, 1), lambda i: (i, 0)),
            pl.BlockSpec((1, NPADN), lambda i: (0, 0)),
            pl.BlockSpec((1000, D), lambda i: (i, 0)),
            pl.BlockSpec((D, 2), lambda i: (0, 0)),
        ],
        out_specs=[
            pl.BlockSpec((1000, 1), lambda i: (i, 0)),
            pl.BlockSpec((1000, 1), lambda i: (i, 0)),
            pl.BlockSpec((1000, 1), lambda i: (i, 0)),
        ],
        out_shape=[
            jax.ShapeDtypeStruct((N, 1), jnp.int32),
            jax.ShapeDtypeStruct((N, 1), jnp.float32),
            jax.ShapeDtypeStruct((N, 1), jnp.float32),
        ],
    )(info_col, info_row, x, att_w)


GROUP = 40
NG = (EPT // 128) // GROUP


def _p3_tile(src_h, dst_h, w_h, nm_h, p_h, q_h, lamb_h,
             m_h, sord_h, skrc_h, set_h, sw_h, keep_h, a_h, b_h,
             nm_v, src_v, dst_v, w_v, p_v, q_v, lamb_v,
             zeros_v, i1_v, f1_v, f2_v, nidx_v, nid_v, na_v, nb_v,
             so_v, sk_v, se_v, sw_v, s_cnt):
    cid = lax.axis_index("c")
    wid = lax.axis_index("s")

    @pl.when(cid == 0)
    def _work():
        i16 = lax.broadcasted_iota(jnp.int32, (16,), 0)

        def _fill(i, _):
            zeros_v[pl.ds(i * 16, 16)] = jnp.zeros((16,), jnp.float32)
            return 0
        lax.fori_loop(0, 512, _fill, 0)

        def _fones(c, _):
            for k in range(8):
                f1_v[pl.ds(c * 128 + k * 16, 16)] = jnp.ones((16,), jnp.float32)
            return 0
        lax.fori_loop(0, GROUP, _fones, 0)
        for k in range(SLOWPT // 16):
            sk_v[pl.ds(k * 16, 16)] = jnp.full((16,), -1, jnp.int32)

        def _zm(i, _):
            pltpu.sync_copy(zeros_v, m_h.at[pl.ds(wid * 262144 + i * 8192, 8192)])
            return 0
        lax.fori_loop(0, 32, _zm, 0)

        def _zc(i, _):
            pltpu.sync_copy(
                zeros_v,
                s_cnt.at[pl.ds(wid * (HASH // 16) + i * 8192, 8192)])
            return 0
        lax.fori_loop(0, HASH // (16 * 8192), _zc, 0)

        pltpu.sync_copy(nm_h, nm_v)
        pltpu.sync_copy(src_h.at[pl.ds(wid * EPT, EPT)], src_v)
        pltpu.sync_copy(dst_h.at[pl.ds(wid * EPT, EPT)], dst_v)
        pltpu.sync_copy(w_h.at[pl.ds(wid * EPT, EPT)], w_v)
        pltpu.sync_copy(p_h.at[pl.ds(wid * NPT, NPT)], p_v)
        pltpu.sync_copy(q_h.at[pl.ds(wid * NPT, NPT)], q_v)
        pltpu.sync_copy(lamb_h, lamb_v)

        plsc.subcore_barrier()

        def _relabel(off16):
            sv = src_v[pl.ds(off16, 16)]
            s16 = sv & 16383
            e16 = sv >> 14
            d16 = dst_v[pl.ds(off16, 16)]
            ns = plsc.load_gather(nm_v, [s16])
            nd = plsc.load_gather(nm_v, [d16])
            ordv = wid * EPT + off16 + i16
            valid = (ns >= 0) & (nd >= 0) & (ordv < E)
            krc = jnp.where(valid, (ns << 11) + nd, -1)
            h = jnp.where(valid, krc & (HASH - 1), HASH + wid * 16 + i16)
            return e16, ordv, valid, krc, h

        for g in range(NG):
            def _ph1(c, _):
                for k in range(8):
                    off16 = (g * GROUP + c) * 128 + k * 16
                    _, _, _, _, h = _relabel(off16)
                    i1_v[pl.ds(c * 128 + k * 16, 16)] = h
                return 0
            lax.fori_loop(0, GROUP, _ph1, 0)
            pltpu.sync_copy(f1_v, s_cnt.at[i1_v], add=True)

        def _node(c, _):
            for k in range(8):
                off16 = c * 128 + k * 16
                nmv = nm_v[pl.ds(wid * NPT + off16, 16)]
                ok = nmv >= 0
                nidx_v[pl.ds(c * 128 + k * 16, 16)] = jnp.where(ok, nmv, NK + wid)
                nid_v[pl.ds(c * 128 + k * 16, 16)] = wid * NPT + off16 + i16
                na_v[pl.ds(c * 128 + k * 16, 16)] = p_v[pl.ds(off16, 16)]
                nb_v[pl.ds(c * 128 + k * 16, 16)] = q_v[pl.ds(off16, 16)]
            return 0
        lax.fori_loop(0, NPT // 128, _node, 0)
        pltpu.sync_copy(nid_v, keep_h.at[nidx_v])
        pltpu.sync_copy(na_v, a_h.at[nidx_v])
        pltpu.sync_copy(nb_v, b_h.at[nidx_v])

        plsc.subcore_barrier()

        scnt = jnp.int32(0)
        for g in range(NG):
            def _ph2a(c, _):
                for k in range(8):
                    off16 = (g * GROUP + c) * 128 + k * 16
                    _, _, _, _, h = _relabel(off16)
                    i1_v[pl.ds(c * 128 + k * 16, 16)] = h
                return 0
            lax.fori_loop(0, GROUP, _ph2a, 0)
            pltpu.sync_copy(s_cnt.at[i1_v], f1_v)

            def _ph2b(c, scnt):
                for k in range(8):
                    off16 = (g * GROUP + c) * 128 + k * 16
                    e16, ordv, valid, krc, _ = _relabel(off16)
                    dup = f1_v[pl.ds(c * 128 + k * 16, 16)] > 1.5
                    fast = valid & jnp.logical_not(dup)
                    i1_v[pl.ds(c * 128 + k * 16, 16)] = jnp.where(
                        fast, krc, CP * CP + wid * 16 + i16)
                    w16 = w_v[pl.ds(off16, 16)]
                    le = plsc.load_gather(lamb_v, [e16])
                    f2_v[pl.ds(c * 128 + k * 16, 16)] = jnp.where(fast, le * w16, 0.0)
                    smask = valid & dup & (scnt < SLOWPT - 16)
                    scl = jnp.minimum(scnt, SLOWPT - 16)
                    plsc.store_compressed(so_v.at[pl.ds(scl, 16)], ordv, mask=smask)
                    plsc.store_compressed(sk_v.at[pl.ds(scl, 16)], krc, mask=smask)
                    plsc.store_compressed(se_v.at[pl.ds(scl, 16)], e16, mask=smask)
                    plsc.store_compressed(sw_v.at[pl.ds(scl, 16)], w16, mask=smask)
                    scnt = scnt + jnp.sum(smask.astype(jnp.int32))
                return scnt
            scnt = lax.fori_loop(0, GROUP, _ph2b, scnt)
            pltpu.sync_copy(f2_v, m_h.at[i1_v])

        pltpu.sync_copy(so_v, sord_h.at[pl.ds(wid * SLOWPT, SLOWPT)])
        pltpu.sync_copy(sk_v, skrc_h.at[pl.ds(wid * SLOWPT, SLOWPT)])
        pltpu.sync_copy(se_v, set_h.at[pl.ds(wid * SLOWPT, SLOWPT)])
        pltpu.sync_copy(sw_v, sw_h.at[pl.ds(wid * SLOWPT, SLOWPT)])


def _p3(src, dst, w, nm_sc, p_sc, q_sc, lamb16):
    mesh = plsc.VectorSubcoreMesh(core_axis_name="c", subcore_axis_name="s")
    f32, i32 = jnp.float32, jnp.int32
    kern = pl.kernel(
        _p3_tile,
        out_type=[
            jax.ShapeDtypeStruct((MSIZE,), f32),
            jax.ShapeDtypeStruct((K,), i32),
            jax.ShapeDtypeStruct((K,), i32),
            jax.ShapeDtypeStruct((K,), i32),
            jax.ShapeDtypeStruct((K,), f32),
            jax.ShapeDtypeStruct((NK + 48,), i32),
            jax.ShapeDtypeStruct((NK + 48,), f32),
            jax.ShapeDtypeStruct((NK + 48,), f32),
        ],
        mesh=mesh,
        compiler_params=pltpu.CompilerParams(needs_layout_passes=False),
        scratch_types=[
            pltpu.VMEM((NPADN,), i32),
            pltpu.VMEM((EPT,), i32),
            pltpu.VMEM((EPT,), i32),
            pltpu.VMEM((EPT,), f32),
            pltpu.VMEM((NPT,), f32),
            pltpu.VMEM((NPT,), f32),
            pltpu.VMEM((16,), f32),
            pltpu.VMEM((8192,), f32),
            pltpu.VMEM((GROUP * 128,), i32),
            pltpu.VMEM((GROUP * 128,), f32),
            pltpu.VMEM((GROUP * 128,), f32),
            pltpu.VMEM((NPT,), i32),
            pltpu.VMEM((NPT,), i32),
            pltpu.VMEM((NPT,), f32),
            pltpu.VMEM((NPT,), f32),
            pltpu.VMEM((SLOWPT,), i32),
            pltpu.VMEM((SLOWPT,), i32),
            pltpu.VMEM((SLOWPT,), i32),
            pltpu.VMEM((SLOWPT,), f32),
            pltpu.VMEM_SHARED((HASH + 512,), f32),
        ],
    )
    return kern(src, dst, w, nm_sc, p_sc, q_sc, lamb16)


def _p4_body(a_col, b_row, m_ref, oc, orr, kc, kr, ec, er, wc, wr, lamb_ref,
             out_ref, resval_scr, z_scr):
    pid = pl.program_id(0)
    f32 = jnp.float32

    @pl.when(pid == 0)
    def _resolve():
        kcol = kc[...]
        ocol = oc[...]
        ecol = ec[...]
        vcol = kcol >= 0
        l0 = lamb_ref[0, 0]
        l1 = lamb_ref[0, 1]
        l2 = lamb_ref[0, 2]
        for j in range(K // 256):
            sl = slice(j * 256, (j + 1) * 256)
            krow = kr[:, sl]
            orow = orr[:, sl]
            erow = er[:, sl]
            wrow = wr[:, sl]
            beat = (vcol & (kcol == krow) & (ecol == erow)
                    & (ocol > orow)).astype(f32)
            lose = jnp.sum(beat, axis=0, keepdims=True) > 0.0
            lrow = jnp.where(erow == 0, l0, jnp.where(erow == 1, l1, l2))
            rv = jnp.where((krow >= 0) & jnp.logical_not(lose),
                           lrow * wrow, 0.0)
            resval_scr[:, sl] = rv

    iidx = pid * 256 + lax.broadcasted_iota(jnp.int32, (256, 1), 0)
    krow_all = kr[...]
    rmatch = ((krow_all >> 11) == iidx).astype(f32)
    amat = rmatch * resval_scr[...]
    kcol_all = kc[...]
    ccol = kcol_all & (CP - 1)
    av = a_col[...]
    bv = b_row[...]
    for j in range(CP // 512):
        sl = slice(j * 512, (j + 1) * 512)
        cidx = j * 512 + lax.broadcasted_iota(jnp.int32, (1, 512), 1)
        bmat = (ccol == cidx).astype(f32)
        fix = lax.dot_general(amat, bmat, (((1,), (0,)), ((), ())),
                              preferred_element_type=f32,
                              precision=lax.Precision.HIGHEST)
        t = av + bv[:, sl]
        wmat = jnp.where(t >= 0, t, NEG * t)
        z_scr[:, sl] = (wmat + m_ref[:, sl]) + fix

    z = z_scr[...]
    rmax = jnp.max(z, axis=1, keepdims=True)
    lo = rmax - 1.0
    hi = rmax

    def _bis(_, carry):
        lo, hi = carry
        mid = 0.5 * (lo + hi)
        fs = jnp.sum(jnp.maximum(z - mid, 0.0), axis=1, keepdims=True)
        big = fs > 1.0
        return jnp.where(big, mid, lo), jnp.where(big, hi, mid)
    lo, hi = lax.fori_loop(0, BISECT_ITERS, _bis, (lo, hi))

    sup = z > lo
    kcount = jnp.sum(sup.astype(f32), axis=1, keepdims=True)
    ssum = jnp.sum(jnp.where(sup, z, 0.0), axis=1, keepdims=True)
    tau = (ssum - 1.0) / kcount
    out_ref[...] = jnp.maximum(z - tau, 0.0)


def _p4(a_col, b_row, m2d, sord, skrc, setype, sw, lamb_row):
    f32 = jnp.float32
    full = lambda r, c: pl.BlockSpec((r, c), lambda i: (0, 0))
    return pl.pallas_call(
        _p4_body,
        grid=(CP // 256,),
        in_specs=[
            pl.BlockSpec((256, 1), lambda i: (i, 0)),
            full(1, CP),
            pl.BlockSpec((256, CP), lambda i: (i, 0)),
            full(K, 1), full(1, K),
            full(K, 1), full(1, K),
            full(K, 1), full(1, K),
            full(K, 1), full(1, K),
            full(1, 128),
        ],
        out_specs=pl.BlockSpec((256, CP), lambda i: (i, 0)),
        out_shape=jax.ShapeDtypeStruct((CP, CP), f32),
        scratch_shapes=[
            pltpu.VMEM((1, K), f32),
            pltpu.VMEM((256, CP), f32),
        ],
    )(a_col, b_row, m2d, sord.reshape(K, 1), sord.reshape(1, K),
      skrc.reshape(K, 1), skrc.reshape(1, K),
      setype.reshape(K, 1), setype.reshape(1, K),
      sw.reshape(K, 1), sw.reshape(1, K), lamb_row)


def kernel(x, edge_index, edge_weights, edge_types, att, lamb):
    f32, i32 = jnp.float32, jnp.int32
    src, dst = edge_index[0], edge_index[1]

    msg = x[src] * edge_weights[:, None]
    deg = jnp.zeros((N,), x.dtype).at[dst].add(1.0)
    h1 = jnp.zeros_like(x).at[dst].add(msg) / jnp.maximum(deg, 1.0)[:, None]
    info = jnp.abs(jnp.sum(x - h1, axis=-1))

    info_pad = jnp.concatenate([info, jnp.full((NPADN - N,), -jnp.inf, f32)])
    att_w = jnp.stack([att[0, :D], att[0, D:]], axis=1)
    nm, p, q = _p1(info_pad, x, att_w)

    nm_sc = jnp.concatenate([nm[:, 0], jnp.full((NPADN - N,), -1, i32)])
    p_sc = jnp.concatenate([p[:, 0], jnp.zeros((NPADN - N,), f32)])
    q_sc = jnp.concatenate([q[:, 0], jnp.zeros((NPADN - N,), f32)])
    zpad_i = jnp.zeros((EPAD - E,), i32)
    src_p = jnp.concatenate([src + (edge_types << 14), zpad_i])
    dst_p = jnp.concatenate([dst, zpad_i])
    w_p = jnp.concatenate([edge_weights, jnp.zeros((EPAD - E,), f32)])
    lamb16 = jnp.concatenate([lamb.reshape(3), jnp.zeros((13,), f32)])

    m_flat, sord, skrc, setype, sw, keep_o, a_o, b_o = _p3(
        src_p, dst_p, w_p, nm_sc, p_sc, q_sc, lamb16)

    m2d = m_flat[:CP * CP].reshape(CP, CP)
    a_col = a_o[:CP].reshape(CP, 1)
    b_row = jnp.concatenate([b_o[:NK], jnp.full((CP - NK,), -1e30, f32)]
                            ).reshape(1, CP)
    lamb_row = jnp.concatenate([lamb.reshape(3), jnp.zeros((125,), f32)]
                               ).reshape(1, 128)

    adj = _p4(a_col, b_row, m2d, sord, skrc, setype, sw, lamb_row)
    return keep_o[:NK], adj[:NK, :NK]


# speedup vs baseline: 5.1545x; 5.1545x over previous
import functools

import jax
import jax.numpy as jnp
from jax import lax
from jax.experimental import pallas as pl
from jax.experimental.pallas import tpu as pltpu
from jax.experimental.pallas import tpu_sc as plsc

N = 10000
E = 320000
D = 512
NK = 2000
NEG = 0.2
CP = 2048
NPADN = 10240
EPAD = 327680
EPT = EPAD // 16
NPT = NPADN // 16
HASH = 1 << 18
SLOWPT = 128
K = 16 * SLOWPT
MSIZE = CP * CP + 16384
BISECT_ITERS = 24


def _p1_body(info_col, info_row, x_ref, att_ref, nm_ref, p_ref, q_ref):
    pid = pl.program_id(0)
    ii = info_col[...]
    iidx = pid * 1000 + lax.broadcasted_iota(jnp.int32, (1000, 1), 0)
    rank = jnp.zeros((1000, 1), jnp.float32)
    for c in range(10):
        jj = info_row[:, c * 1024:(c + 1) * 1024]
        jidx = c * 1024 + lax.broadcasted_iota(jnp.int32, (1, 1024), 1)
        gt = (jj > ii).astype(jnp.float32)
        tie = ((jj == ii) & (jidx < iidx)).astype(jnp.float32)
        rank = rank + jnp.sum(gt + tie, axis=1, keepdims=True)
    r32 = rank.astype(jnp.int32)
    nm_ref[...] = jnp.where(r32 < NK, r32, -1)
    pq = lax.dot_general(x_ref[...], att_ref[...], (((1,), (0,)), ((), ())),
                         preferred_element_type=jnp.float32)
    p_ref[...] = pq[:, 0:1]
    q_ref[...] = pq[:, 1:2]


def _p1(info_pad, x, att_w):
    info_col = info_pad.reshape(NPADN, 1)
    info_row = info_pad.reshape(1, NPADN)
    return pl.pallas_call(
        _p1_body,
        grid=(10,),
        in_specs=[
            pl.BlockSpec((1000, 1), lambda i: (i, 0)),
            pl.BlockSpec((1, NPADN), lambda i: (0, 0)),
            pl.BlockSpec((1000, D), lambda i: (i, 0)),
            pl.BlockSpec((D, 2), lambda i: (0, 0)),
        ],
        out_specs=[
            pl.BlockSpec((1000, 1), lambda i: (i, 0)),
            pl.BlockSpec((1000, 1), lambda i: (i, 0)),
            pl.BlockSpec((1000, 1), lambda i: (i, 0)),
        ],
        out_shape=[
            jax.ShapeDtypeStruct((N, 1), jnp.int32),
            jax.ShapeDtypeStruct((N, 1), jnp.float32),
            jax.ShapeDtypeStruct((N, 1), jnp.float32),
        ],
    )(info_col, info_row, x, att_w)


GROUP = 40
NG = (EPT // 128) // GROUP


def _p3_tile(src_h, dst_h, w_h, nm_h, p_h, q_h, lamb_h,
             m_h, sord_h, skrc_h, set_h, sw_h, keep_h, a_h, b_h,
             nm_v, src_v, dst_v, w_v, p_v, q_v, lamb_v,
             zeros_v, i1_v, f1_v, f2_v, nidx_v, nid_v, na_v, nb_v,
             so_v, sk_v, se_v, sw_v, s_cnt):
    cid = lax.axis_index("c")
    wid = lax.axis_index("s")

    @pl.when(cid == 0)
    def _work():
        i16 = lax.broadcasted_iota(jnp.int32, (16,), 0)

        def _fill(i, _):
            zeros_v[pl.ds(i * 16, 16)] = jnp.zeros((16,), jnp.float32)
            return 0
        lax.fori_loop(0, 512, _fill, 0)

        def _fones(c, _):
            for k in range(8):
                f1_v[pl.ds(c * 128 + k * 16, 16)] = jnp.ones((16,), jnp.float32)
            return 0
        lax.fori_loop(0, GROUP, _fones, 0)
        for k in range(SLOWPT // 16):
            sk_v[pl.ds(k * 16, 16)] = jnp.full((16,), -1, jnp.int32)

        def _zm(i, _):
            pltpu.sync_copy(zeros_v, m_h.at[pl.ds(wid * 262144 + i * 8192, 8192)])
            return 0
        lax.fori_loop(0, 32, _zm, 0)

        def _zc(i, _):
            pltpu.sync_copy(
                zeros_v,
                s_cnt.at[pl.ds(wid * (HASH // 16) + i * 8192, 8192)])
            return 0
        lax.fori_loop(0, HASH // (16 * 8192), _zc, 0)

        pltpu.sync_copy(nm_h, nm_v)
        pltpu.sync_copy(src_h.at[pl.ds(wid * EPT, EPT)], src_v)
        pltpu.sync_copy(dst_h.at[pl.ds(wid * EPT, EPT)], dst_v)
        pltpu.sync_copy(w_h.at[pl.ds(wid * EPT, EPT)], w_v)
        pltpu.sync_copy(p_h.at[pl.ds(wid * NPT, NPT)], p_v)
        pltpu.sync_copy(q_h.at[pl.ds(wid * NPT, NPT)], q_v)
        pltpu.sync_copy(lamb_h, lamb_v)

        plsc.subcore_barrier()

        def _relabel(off16):
            sv = src_v[pl.ds(off16, 16)]
            s16 = sv & 16383
            e16 = sv >> 14
            d16 = dst_v[pl.ds(off16, 16)]
            ns = plsc.load_gather(nm_v, [s16])
            nd = plsc.load_gather(nm_v, [d16])
            ordv = wid * EPT + off16 + i16
            valid = (ns >= 0) & (nd >= 0) & (ordv < E)
            krc = jnp.where(valid, (ns << 11) + nd, -1)
            h = jnp.where(valid, krc & (HASH - 1), HASH + (ordv & 16383))
            return e16, ordv, valid, krc, h

        for g in range(NG):
            def _ph1(c, _):
                for k in range(8):
                    off16 = (g * GROUP + c) * 128 + k * 16
                    _, _, _, _, h = _relabel(off16)
                    i1_v[pl.ds(c * 128 + k * 16, 16)] = h
                return 0
            lax.fori_loop(0, GROUP, _ph1, 0)
            pltpu.sync_copy(f1_v, s_cnt.at[i1_v], add=True)

        def _node(c, _):
            for k in range(8):
                off16 = c * 128 + k * 16
                nmv = nm_v[pl.ds(wid * NPT + off16, 16)]
                ok = nmv >= 0
                nidx_v[pl.ds(c * 128 + k * 16, 16)] = jnp.where(
                    ok, nmv, NK + ((wid * NPT + off16 + i16) & 2047))
                nid_v[pl.ds(c * 128 + k * 16, 16)] = wid * NPT + off16 + i16
                na_v[pl.ds(c * 128 + k * 16, 16)] = p_v[pl.ds(off16, 16)]
                nb_v[pl.ds(c * 128 + k * 16, 16)] = q_v[pl.ds(off16, 16)]
            return 0
        lax.fori_loop(0, NPT // 128, _node, 0)
        pltpu.sync_copy(nid_v, keep_h.at[nidx_v])
        pltpu.sync_copy(na_v, a_h.at[nidx_v])
        pltpu.sync_copy(nb_v, b_h.at[nidx_v])

        plsc.subcore_barrier()

        scnt = jnp.int32(0)
        for g in range(NG):
            def _ph2a(c, _):
                for k in range(8):
                    off16 = (g * GROUP + c) * 128 + k * 16
                    _, _, _, _, h = _relabel(off16)
                    i1_v[pl.ds(c * 128 + k * 16, 16)] = h
                return 0
            lax.fori_loop(0, GROUP, _ph2a, 0)
            pltpu.sync_copy(s_cnt.at[i1_v], f1_v)

            def _ph2b(c, scnt):
                for k in range(8):
                    off16 = (g * GROUP + c) * 128 + k * 16
                    e16, ordv, valid, krc, _ = _relabel(off16)
                    dup = f1_v[pl.ds(c * 128 + k * 16, 16)] > 1.5
                    fast = valid & jnp.logical_not(dup)
                    i1_v[pl.ds(c * 128 + k * 16, 16)] = jnp.where(
                        fast, krc, CP * CP + (ordv & 16383))
                    w16 = w_v[pl.ds(off16, 16)]
                    le = plsc.load_gather(lamb_v, [e16])
                    f2_v[pl.ds(c * 128 + k * 16, 16)] = jnp.where(fast, le * w16, 0.0)
                    smask = valid & dup & (scnt < SLOWPT - 16)
                    scl = jnp.minimum(scnt, SLOWPT - 16)
                    plsc.store_compressed(so_v.at[pl.ds(scl, 16)], ordv, mask=smask)
                    plsc.store_compressed(sk_v.at[pl.ds(scl, 16)], krc, mask=smask)
                    plsc.store_compressed(se_v.at[pl.ds(scl, 16)], e16, mask=smask)
                    plsc.store_compressed(sw_v.at[pl.ds(scl, 16)], w16, mask=smask)
                    scnt = scnt + jnp.sum(smask.astype(jnp.int32))
                return scnt
            scnt = lax.fori_loop(0, GROUP, _ph2b, scnt)
            pltpu.sync_copy(f2_v, m_h.at[i1_v])

        pltpu.sync_copy(so_v, sord_h.at[pl.ds(wid * SLOWPT, SLOWPT)])
        pltpu.sync_copy(sk_v, skrc_h.at[pl.ds(wid * SLOWPT, SLOWPT)])
        pltpu.sync_copy(se_v, set_h.at[pl.ds(wid * SLOWPT, SLOWPT)])
        pltpu.sync_copy(sw_v, sw_h.at[pl.ds(wid * SLOWPT, SLOWPT)])


def _p3(src, dst, w, nm_sc, p_sc, q_sc, lamb16):
    mesh = plsc.VectorSubcoreMesh(core_axis_name="c", subcore_axis_name="s")
    f32, i32 = jnp.float32, jnp.int32
    kern = pl.kernel(
        _p3_tile,
        out_type=[
            jax.ShapeDtypeStruct((MSIZE,), f32),
            jax.ShapeDtypeStruct((K,), i32),
            jax.ShapeDtypeStruct((K,), i32),
            jax.ShapeDtypeStruct((K,), i32),
            jax.ShapeDtypeStruct((K,), f32),
            jax.ShapeDtypeStruct((NK + 2048,), i32),
            jax.ShapeDtypeStruct((NK + 2048,), f32),
            jax.ShapeDtypeStruct((NK + 2048,), f32),
        ],
        mesh=mesh,
        compiler_params=pltpu.CompilerParams(needs_layout_passes=False),
        scratch_types=[
            pltpu.VMEM((NPADN,), i32),
            pltpu.VMEM((EPT,), i32),
            pltpu.VMEM((EPT,), i32),
            pltpu.VMEM((EPT,), f32),
            pltpu.VMEM((NPT,), f32),
            pltpu.VMEM((NPT,), f32),
            pltpu.VMEM((16,), f32),
            pltpu.VMEM((8192,), f32),
            pltpu.VMEM((GROUP * 128,), i32),
            pltpu.VMEM((GROUP * 128,), f32),
            pltpu.VMEM((GROUP * 128,), f32),
            pltpu.VMEM((NPT,), i32),
            pltpu.VMEM((NPT,), i32),
            pltpu.VMEM((NPT,), f32),
            pltpu.VMEM((NPT,), f32),
            pltpu.VMEM((SLOWPT,), i32),
            pltpu.VMEM((SLOWPT,), i32),
            pltpu.VMEM((SLOWPT,), i32),
            pltpu.VMEM((SLOWPT,), f32),
            pltpu.VMEM_SHARED((HASH + 16384,), f32),
        ],
    )
    return kern(src, dst, w, nm_sc, p_sc, q_sc, lamb16)


def _p4_body(a_col, b_row, m_ref, oc, orr, kc, kr, ec, er, wc, wr, lamb_ref,
             out_ref, resval_scr, z_scr):
    pid = pl.program_id(0)
    f32 = jnp.float32

    @pl.when(pid == 0)
    def _resolve():
        kcol = kc[...]
        ocol = oc[...]
        ecol = ec[...]
        vcol = kcol >= 0
        l0 = lamb_ref[0, 0]
        l1 = lamb_ref[0, 1]
        l2 = lamb_ref[0, 2]
        for j in range(K // 256):
            sl = slice(j * 256, (j + 1) * 256)
            krow = kr[:, sl]
            orow = orr[:, sl]
            erow = er[:, sl]
            wrow = wr[:, sl]
            beat = (vcol & (kcol == krow) & (ecol == erow)
                    & (ocol > orow)).astype(f32)
            lose = jnp.sum(beat, axis=0, keepdims=True) > 0.0
            lrow = jnp.where(erow == 0, l0, jnp.where(erow == 1, l1, l2))
            rv = jnp.where((krow >= 0) & jnp.logical_not(lose),
                           lrow * wrow, 0.0)
            resval_scr[:, sl] = rv

    iidx = pid * 256 + lax.broadcasted_iota(jnp.int32, (256, 1), 0)
    krow_all = kr[...]
    rmatch = ((krow_all >> 11) == iidx).astype(f32)
    amat = rmatch * resval_scr[...]
    kcol_all = kc[...]
    ccol = kcol_all & (CP - 1)
    av = a_col[...]
    bv = b_row[...]
    for j in range(CP // 512):
        sl = slice(j * 512, (j + 1) * 512)
        cidx = j * 512 + lax.broadcasted_iota(jnp.int32, (1, 512), 1)
        bmat = (ccol == cidx).astype(f32)
        fix = lax.dot_general(amat, bmat, (((1,), (0,)), ((), ())),
                              preferred_element_type=f32,
                              precision=lax.Precision.HIGHEST)
        t = av + bv[:, sl]
        wmat = jnp.where(t >= 0, t, NEG * t)
        z_scr[:, sl] = (wmat + m_ref[:, sl]) + fix

    z = z_scr[...]
    rmax = jnp.max(z, axis=1, keepdims=True)
    lo = rmax - 1.0
    hi = rmax

    def _bis(_, carry):
        lo, hi = carry
        mid = 0.5 * (lo + hi)
        fs = jnp.sum(jnp.maximum(z - mid, 0.0), axis=1, keepdims=True)
        big = fs > 1.0
        return jnp.where(big, mid, lo), jnp.where(big, hi, mid)
    lo, hi = lax.fori_loop(0, BISECT_ITERS, _bis, (lo, hi))

    sup = z > lo
    kcount = jnp.sum(sup.astype(f32), axis=1, keepdims=True)
    ssum = jnp.sum(jnp.where(sup, z, 0.0), axis=1, keepdims=True)
    tau = (ssum - 1.0) / kcount
    out_ref[...] = jnp.maximum(z - tau, 0.0)


def _p4(a_col, b_row, m2d, sord, skrc, setype, sw, lamb_row):
    f32 = jnp.float32
    full = lambda r, c: pl.BlockSpec((r, c), lambda i: (0, 0))
    return pl.pallas_call(
        _p4_body,
        grid=(CP // 256,),
        in_specs=[
            pl.BlockSpec((256, 1), lambda i: (i, 0)),
            full(1, CP),
            pl.BlockSpec((256, CP), lambda i: (i, 0)),
            full(K, 1), full(1, K),
            full(K, 1), full(1, K),
            full(K, 1), full(1, K),
            full(K, 1), full(1, K),
            full(1, 128),
        ],
        out_specs=pl.BlockSpec((256, CP), lambda i: (i, 0)),
        out_shape=jax.ShapeDtypeStruct((CP, CP), f32),
        scratch_shapes=[
            pltpu.VMEM((1, K), f32),
            pltpu.VMEM((256, CP), f32),
        ],
    )(a_col, b_row, m2d, sord.reshape(K, 1), sord.reshape(1, K),
      skrc.reshape(K, 1), skrc.reshape(1, K),
      setype.reshape(K, 1), setype.reshape(1, K),
      sw.reshape(K, 1), sw.reshape(1, K), lamb_row)


def kernel(x, edge_index, edge_weights, edge_types, att, lamb):
    f32, i32 = jnp.float32, jnp.int32
    src, dst = edge_index[0], edge_index[1]

    msg = x[src] * edge_weights[:, None]
    deg = jnp.zeros((N,), x.dtype).at[dst].add(1.0)
    h1 = jnp.zeros_like(x).at[dst].add(msg) / jnp.maximum(deg, 1.0)[:, None]
    info = jnp.abs(jnp.sum(x - h1, axis=-1))

    info_pad = jnp.concatenate([info, jnp.full((NPADN - N,), -jnp.inf, f32)])
    att_w = jnp.stack([att[0, :D], att[0, D:]], axis=1)
    nm, p, q = _p1(info_pad, x, att_w)

    nm_sc = jnp.concatenate([nm[:, 0], jnp.full((NPADN - N,), -1, i32)])
    p_sc = jnp.concatenate([p[:, 0], jnp.zeros((NPADN - N,), f32)])
    q_sc = jnp.concatenate([q[:, 0], jnp.zeros((NPADN - N,), f32)])
    zpad_i = jnp.zeros((EPAD - E,), i32)
    src_p = jnp.concatenate([src + (edge_types << 14), zpad_i])
    dst_p = jnp.concatenate([dst, zpad_i])
    w_p = jnp.concatenate([edge_weights, jnp.zeros((EPAD - E,), f32)])
    lamb16 = jnp.concatenate([lamb.reshape(3), jnp.zeros((13,), f32)])

    m_flat, sord, skrc, setype, sw, keep_o, a_o, b_o = _p3(
        src_p, dst_p, w_p, nm_sc, p_sc, q_sc, lamb16)

    m2d = m_flat[:CP * CP].reshape(CP, CP)
    a_col = a_o[:CP].reshape(CP, 1)
    b_row = jnp.concatenate([b_o[:NK], jnp.full((CP - NK,), -1e30, f32)]
                            ).reshape(1, CP)
    lamb_row = jnp.concatenate([lamb.reshape(3), jnp.zeros((125,), f32)]
                               ).reshape(1, 128)

    adj = _p4(a_col, b_row, m2d, sord, skrc, setype, sw, lamb_row)
    return keep_o[:NK], adj[:NK, :NK]


# speedup vs baseline: 5.1577x; 1.0006x over previous
import jax
import jax.numpy as jnp
from jax import lax
from jax.experimental import pallas as pl
from jax.experimental.pallas import tpu as pltpu
from jax.experimental.pallas import tpu_sc as plsc

N = 10000
E = 320000
D = 512
NK = 2000
NEG = 0.2
CP = 2048
NPADN = 10240
EPAD = 327680
EPT = EPAD // 16
NPT = NPADN // 16
HASH = 1 << 18
SLOWPT = 128
K = 16 * SLOWPT
MSIZE = CP * CP + 16384
BISECT_ITERS = 24


def _p1_body(info_col, info_row, x_ref, att_ref, nm_ref, p_ref, q_ref):
    pid = pl.program_id(0)
    ii = info_col[...]
    iidx = pid * 1000 + lax.broadcasted_iota(jnp.int32, (1000, 1), 0)
    rank = jnp.zeros((1000, 1), jnp.float32)
    for c in range(10):
        jj = info_row[:, c * 1024:(c + 1) * 1024]
        jidx = c * 1024 + lax.broadcasted_iota(jnp.int32, (1, 1024), 1)
        gt = (jj > ii).astype(jnp.float32)
        tie = ((jj == ii) & (jidx < iidx)).astype(jnp.float32)
        rank = rank + jnp.sum(gt + tie, axis=1, keepdims=True)
    r32 = rank.astype(jnp.int32)
    nm_ref[...] = jnp.where(r32 < NK, r32, -1)
    pq = lax.dot_general(x_ref[...], att_ref[...], (((1,), (0,)), ((), ())),
                         preferred_element_type=jnp.float32)
    p_ref[...] = pq[:, 0:1]
    q_ref[...] = pq[:, 1:2]


def _p1(info_pad, x, att_w):
    info_col = info_pad.reshape(NPADN, 1)
    info_row = info_pad.reshape(1, NPADN)
    return pl.pallas_call(
        _p1_body,
        grid=(10,),
        in_specs=[
            pl.BlockSpec((1000, 1), lambda i: (i, 0)),
            pl.BlockSpec((1, NPADN), lambda i: (0, 0)),
            pl.BlockSpec((1000, D), lambda i: (i, 0)),
            pl.BlockSpec((D, 2), lambda i: (0, 0)),
        ],
        out_specs=[
            pl.BlockSpec((1000, 1), lambda i: (i, 0)),
            pl.BlockSpec((1000, 1), lambda i: (i, 0)),
            pl.BlockSpec((1000, 1), lambda i: (i, 0)),
        ],
        out_shape=[
            jax.ShapeDtypeStruct((N, 1), jnp.int32),
            jax.ShapeDtypeStruct((N, 1), jnp.float32),
            jax.ShapeDtypeStruct((N, 1), jnp.float32),
        ],
    )(info_col, info_row, x, att_w)


GROUP = 40
NG = (EPT // 128) // GROUP


def _p3_tile(src_h, dst_h, w_h, nm_h, p_h, q_h, lamb_h,
             m_h, sord_h, skrc_h, set_h, sw_h, keep_h, a_h, b_h,
             nm_v, src_v, dst_v, w_v, p_v, q_v, lamb_v,
             zeros_v, i1_v, f1_v, f2_v, nidx_v, nid_v, na_v, nb_v,
             so_v, sk_v, se_v, sw_v, s_cnt):
    cid = lax.axis_index("c")
    wid = lax.axis_index("s")

    @pl.when(cid == 0)
    def _work():
        i16 = lax.broadcasted_iota(jnp.int32, (16,), 0)

        def _fill(i, _):
            zeros_v[pl.ds(i * 16, 16)] = jnp.zeros((16,), jnp.float32)
            return 0
        lax.fori_loop(0, 512, _fill, 0)

        def _fones(c, _):
            for k in range(8):
                f1_v[pl.ds(c * 128 + k * 16, 16)] = jnp.ones((16,), jnp.float32)
            return 0
        lax.fori_loop(0, GROUP, _fones, 0)
        for k in range(SLOWPT // 16):
            sk_v[pl.ds(k * 16, 16)] = jnp.full((16,), -1, jnp.int32)

        def _zm(i, _):
            pltpu.sync_copy(zeros_v, m_h.at[pl.ds(wid * 262144 + i * 8192, 8192)])
            return 0
        lax.fori_loop(0, 32, _zm, 0)

        def _zc(i, _):
            pltpu.sync_copy(
                zeros_v,
                s_cnt.at[pl.ds(wid * (HASH // 16) + i * 8192, 8192)])
            return 0
        lax.fori_loop(0, HASH // (16 * 8192), _zc, 0)

        pltpu.sync_copy(nm_h, nm_v)
        pltpu.sync_copy(src_h.at[pl.ds(wid * EPT, EPT)], src_v)
        pltpu.sync_copy(dst_h.at[pl.ds(wid * EPT, EPT)], dst_v)
        pltpu.sync_copy(w_h.at[pl.ds(wid * EPT, EPT)], w_v)
        pltpu.sync_copy(p_h.at[pl.ds(wid * NPT, NPT)], p_v)
        pltpu.sync_copy(q_h.at[pl.ds(wid * NPT, NPT)], q_v)
        pltpu.sync_copy(lamb_h, lamb_v)

        plsc.subcore_barrier()

        def _relabel(off16):
            sv = src_v[pl.ds(off16, 16)]
            s16 = sv & 16383
            e16 = sv >> 14
            d16 = dst_v[pl.ds(off16, 16)]
            ns = plsc.load_gather(nm_v, [s16])
            nd = plsc.load_gather(nm_v, [d16])
            ordv = wid * EPT + off16 + i16
            valid = (ns >= 0) & (nd >= 0) & (ordv < E)
            krc = jnp.where(valid, (ns << 11) + nd, -1)
            h = jnp.where(valid, krc & (HASH - 1), HASH + (ordv & 16383))
            return e16, ordv, valid, krc, h

        for g in range(NG):
            def _ph1(c, _):
                for k in range(8):
                    off16 = (g * GROUP + c) * 128 + k * 16
                    _, _, _, _, h = _relabel(off16)
                    i1_v[pl.ds(c * 128 + k * 16, 16)] = h
                return 0
            lax.fori_loop(0, GROUP, _ph1, 0)
            pltpu.sync_copy(f1_v, s_cnt.at[i1_v], add=True)

        def _node(c, _):
            for k in range(8):
                off16 = c * 128 + k * 16
                nmv = nm_v[pl.ds(wid * NPT + off16, 16)]
                ok = nmv >= 0
                nidx_v[pl.ds(c * 128 + k * 16, 16)] = jnp.where(
                    ok, nmv, NK + ((wid * NPT + off16 + i16) & 2047))
                nid_v[pl.ds(c * 128 + k * 16, 16)] = wid * NPT + off16 + i16
                na_v[pl.ds(c * 128 + k * 16, 16)] = p_v[pl.ds(off16, 16)]
                nb_v[pl.ds(c * 128 + k * 16, 16)] = q_v[pl.ds(off16, 16)]
            return 0
        lax.fori_loop(0, NPT // 128, _node, 0)
        pltpu.sync_copy(nid_v, keep_h.at[nidx_v])
        pltpu.sync_copy(na_v, a_h.at[nidx_v])
        pltpu.sync_copy(nb_v, b_h.at[nidx_v])

        plsc.subcore_barrier()

        scnt = jnp.int32(0)
        for g in range(NG):
            def _ph2a(c, _):
                for k in range(8):
                    off16 = (g * GROUP + c) * 128 + k * 16
                    _, _, _, _, h = _relabel(off16)
                    i1_v[pl.ds(c * 128 + k * 16, 16)] = h
                return 0
            lax.fori_loop(0, GROUP, _ph2a, 0)
            pltpu.sync_copy(s_cnt.at[i1_v], f1_v)

            def _ph2b(c, scnt):
                for k in range(8):
                    off16 = (g * GROUP + c) * 128 + k * 16
                    e16, ordv, valid, krc, _ = _relabel(off16)
                    dup = f1_v[pl.ds(c * 128 + k * 16, 16)] > 1.5
                    fast = valid & jnp.logical_not(dup)
                    i1_v[pl.ds(c * 128 + k * 16, 16)] = jnp.where(
                        fast, krc, CP * CP + (ordv & 16383))
                    w16 = w_v[pl.ds(off16, 16)]
                    le = plsc.load_gather(lamb_v, [e16])
                    f2_v[pl.ds(c * 128 + k * 16, 16)] = jnp.where(fast, le * w16, 0.0)
                    smask = valid & dup & (scnt < SLOWPT - 16)
                    scl = jnp.minimum(scnt, SLOWPT - 16)
                    plsc.store_compressed(so_v.at[pl.ds(scl, 16)], ordv, mask=smask)
                    plsc.store_compressed(sk_v.at[pl.ds(scl, 16)], krc, mask=smask)
                    plsc.store_compressed(se_v.at[pl.ds(scl, 16)], e16, mask=smask)
                    plsc.store_compressed(sw_v.at[pl.ds(scl, 16)], w16, mask=smask)
                    scnt = scnt + jnp.sum(smask.astype(jnp.int32))
                return scnt
            scnt = lax.fori_loop(0, GROUP, _ph2b, scnt)
            pltpu.sync_copy(f2_v, m_h.at[i1_v])

        pltpu.sync_copy(so_v, sord_h.at[pl.ds(wid * SLOWPT, SLOWPT)])
        pltpu.sync_copy(sk_v, skrc_h.at[pl.ds(wid * SLOWPT, SLOWPT)])
        pltpu.sync_copy(se_v, set_h.at[pl.ds(wid * SLOWPT, SLOWPT)])
        pltpu.sync_copy(sw_v, sw_h.at[pl.ds(wid * SLOWPT, SLOWPT)])


def _p3(src, dst, w, nm_sc, p_sc, q_sc, lamb16):
    mesh = plsc.VectorSubcoreMesh(core_axis_name="c", subcore_axis_name="s")
    f32, i32 = jnp.float32, jnp.int32
    kern = pl.kernel(
        _p3_tile,
        out_type=[
            jax.ShapeDtypeStruct((MSIZE,), f32),
            jax.ShapeDtypeStruct((K,), i32),
            jax.ShapeDtypeStruct((K,), i32),
            jax.ShapeDtypeStruct((K,), i32),
            jax.ShapeDtypeStruct((K,), f32),
            jax.ShapeDtypeStruct((NK + 2048,), i32),
            jax.ShapeDtypeStruct((NK + 2048,), f32),
            jax.ShapeDtypeStruct((NK + 2048,), f32),
        ],
        mesh=mesh,
        compiler_params=pltpu.CompilerParams(needs_layout_passes=False),
        scratch_types=[
            pltpu.VMEM((NPADN,), i32),
            pltpu.VMEM((EPT,), i32),
            pltpu.VMEM((EPT,), i32),
            pltpu.VMEM((EPT,), f32),
            pltpu.VMEM((NPT,), f32),
            pltpu.VMEM((NPT,), f32),
            pltpu.VMEM((16,), f32),
            pltpu.VMEM((8192,), f32),
            pltpu.VMEM((GROUP * 128,), i32),
            pltpu.VMEM((GROUP * 128,), f32),
            pltpu.VMEM((GROUP * 128,), f32),
            pltpu.VMEM((NPT,), i32),
            pltpu.VMEM((NPT,), i32),
            pltpu.VMEM((NPT,), f32),
            pltpu.VMEM((NPT,), f32),
            pltpu.VMEM((SLOWPT,), i32),
            pltpu.VMEM((SLOWPT,), i32),
            pltpu.VMEM((SLOWPT,), i32),
            pltpu.VMEM((SLOWPT,), f32),
            pltpu.VMEM_SHARED((HASH + 16384,), f32),
        ],
    )
    return kern(src, dst, w, nm_sc, p_sc, q_sc, lamb16)


def _p4_body(a_col, b_row, m_ref, oc, orr, kc, kr, ec, er, wc, wr, lamb_ref,
             out_ref, resval_scr, z_scr):
    pid = pl.program_id(0)
    f32 = jnp.float32

    @pl.when(pid == 0)
    def _resolve():
        kcol = kc[...]
        ocol = oc[...]
        ecol = ec[...]
        vcol = kcol >= 0
        l0 = lamb_ref[0, 0]
        l1 = lamb_ref[0, 1]
        l2 = lamb_ref[0, 2]
        for j in range(K // 256):
            sl = slice(j * 256, (j + 1) * 256)
            krow = kr[:, sl]
            orow = orr[:, sl]
            erow = er[:, sl]
            wrow = wr[:, sl]
            beat = (vcol & (kcol == krow) & (ecol == erow)
                    & (ocol > orow)).astype(f32)
            lose = jnp.sum(beat, axis=0, keepdims=True) > 0.0
            lrow = jnp.where(erow == 0, l0, jnp.where(erow == 1, l1, l2))
            rv = jnp.where((krow >= 0) & jnp.logical_not(lose),
                           lrow * wrow, 0.0)
            resval_scr[:, sl] = rv

    iidx = pid * 256 + lax.broadcasted_iota(jnp.int32, (256, 1), 0)
    krow_all = kr[...]
    rmatch = ((krow_all >> 11) == iidx).astype(f32)
    amat = rmatch * resval_scr[...]
    kcol_all = kc[...]
    ccol = kcol_all & (CP - 1)
    av = a_col[...]
    bv = b_row[...]
    for j in range(CP // 512):
        sl = slice(j * 512, (j + 1) * 512)
        cidx = j * 512 + lax.broadcasted_iota(jnp.int32, (1, 512), 1)
        bmat = (ccol == cidx).astype(f32)
        fix = lax.dot_general(amat, bmat, (((1,), (0,)), ((), ())),
                              preferred_element_type=f32,
                              precision=lax.Precision.HIGHEST)
        t = av + bv[:, sl]
        wmat = jnp.where(t >= 0, t, NEG * t)
        z_scr[:, sl] = (wmat + m_ref[:, sl]) + fix

    z = z_scr[...]
    rmax = jnp.max(z, axis=1, keepdims=True)
    lo = rmax - 1.0
    hi = rmax

    def _bis(_, carry):
        lo, hi = carry
        mid = 0.5 * (lo + hi)
        fs = jnp.sum(jnp.maximum(z - mid, 0.0), axis=1, keepdims=True)
        big = fs > 1.0
        return jnp.where(big, mid, lo), jnp.where(big, hi, mid)
    lo, hi = lax.fori_loop(0, BISECT_ITERS, _bis, (lo, hi))

    sup = z > lo
    kcount = jnp.sum(sup.astype(f32), axis=1, keepdims=True)
    ssum = jnp.sum(jnp.where(sup, z, 0.0), axis=1, keepdims=True)
    tau = (ssum - 1.0) / kcount
    out_ref[...] = jnp.maximum(z - tau, 0.0)


def _p4(a_col, b_row, m2d, sord, skrc, setype, sw, lamb_row):
    f32 = jnp.float32
    full = lambda r, c: pl.BlockSpec((r, c), lambda i: (0, 0))
    return pl.pallas_call(
        _p4_body,
        grid=(CP // 256,),
        in_specs=[
            pl.BlockSpec((256, 1), lambda i: (i, 0)),
            full(1, CP),
            pl.BlockSpec((256, CP), lambda i: (i, 0)),
            full(K, 1), full(1, K),
            full(K, 1), full(1, K),
            full(K, 1), full(1, K),
            full(K, 1), full(1, K),
            full(1, 128),
        ],
        out_specs=pl.BlockSpec((256, CP), lambda i: (i, 0)),
        out_shape=jax.ShapeDtypeStruct((CP, CP), f32),
        scratch_shapes=[
            pltpu.VMEM((1, K), f32),
            pltpu.VMEM((256, CP), f32),
        ],
    )(a_col, b_row, m2d, sord.reshape(K, 1), sord.reshape(1, K),
      skrc.reshape(K, 1), skrc.reshape(1, K),
      setype.reshape(K, 1), setype.reshape(1, K),
      sw.reshape(K, 1), sw.reshape(1, K), lamb_row)


def kernel(x, edge_index, edge_weights, edge_types, att, lamb):
    f32, i32 = jnp.float32, jnp.int32
    src, dst = edge_index[0], edge_index[1]

    msg = x[src] * edge_weights[:, None]
    deg = jnp.zeros((N,), x.dtype).at[dst].add(1.0)
    h1 = jnp.zeros_like(x).at[dst].add(msg) / jnp.maximum(deg, 1.0)[:, None]
    info = jnp.abs(jnp.sum(x - h1, axis=-1))

    info_pad = jnp.concatenate([info, jnp.full((NPADN - N,), -jnp.inf, f32)])
    att_w = jnp.stack([att[0, :D], att[0, D:]], axis=1)
    nm, p, q = _p1(info_pad, x, att_w)

    nm_sc = jnp.concatenate([nm[:, 0], jnp.full((NPADN - N,), -1, i32)])
    p_sc = jnp.concatenate([p[:, 0], jnp.zeros((NPADN - N,), f32)])
    q_sc = jnp.concatenate([q[:, 0], jnp.zeros((NPADN - N,), f32)])
    zpad_i = jnp.zeros((EPAD - E,), i32)
    src_p = jnp.concatenate([src + (edge_types << 14), zpad_i])
    dst_p = jnp.concatenate([dst, zpad_i])
    w_p = jnp.concatenate([edge_weights, jnp.zeros((EPAD - E,), f32)])
    lamb16 = jnp.concatenate([lamb.reshape(3), jnp.zeros((13,), f32)])

    m_flat, sord, skrc, setype, sw, keep_o, a_o, b_o = _p3(
        src_p, dst_p, w_p, nm_sc, p_sc, q_sc, lamb16)

    m2d = m_flat[:CP * CP].reshape(CP, CP)
    a_col = a_o[:CP].reshape(CP, 1)
    b_row = jnp.concatenate([b_o[:NK], jnp.full((CP - NK,), -1e30, f32)]
                            ).reshape(1, CP)
    lamb_row = jnp.concatenate([lamb.reshape(3), jnp.zeros((125,), f32)]
                               ).reshape(1, 128)

    adj = _p4(a_col, b_row, m2d, sord, skrc, setype, sw, lamb_row)
    return keep_o[:NK], adj[:NK, :NK]
